# Initial kernel scaffold; baseline (speedup 1.0000x reference)
import jax
import jax.numpy as jnp
from jax.experimental import pallas as pl


def kernel(byte_ids, logits):
    raise NotImplementedError("write your pallas kernel here")


# speedup vs baseline: 27.7470x; 27.7470x over previous
import functools

import jax
import jax.numpy as jnp
from jax import lax
from jax.experimental import pallas as pl
from jax.experimental.pallas import tpu as pltpu
from jax.experimental.pallas import tpu_sc as plsc

_LANES = 16
_NUM_CORES = 2
_NUM_SUBCORES = 16
_NUM_WORKERS = _NUM_CORES * _NUM_SUBCORES


def _lookup_call(bids_flat, logits_pad, rows_pad, vocab, max_id):
    n = bids_flat.shape[0]
    per_worker = n // _NUM_WORKERS
    rows_per_sub = rows_pad // _NUM_SUBCORES
    row_groups = rows_per_sub // _LANES
    n_idx_vecs = per_worker // _LANES

    mesh = plsc.VectorSubcoreMesh(core_axis_name="c", subcore_axis_name="s",
                                  num_cores=_NUM_CORES,
                                  num_subcores=_NUM_SUBCORES)

    @functools.partial(
        pl.kernel,
        mesh=mesh,
        out_type=jax.ShapeDtypeStruct((n,), jnp.int32),
        compiler_params=pltpu.CompilerParams(needs_layout_passes=False),
        scratch_types=[
            pltpu.VMEM((rows_per_sub * vocab,), jnp.float32),
            pltpu.VMEM((per_worker,), jnp.int32),
            pltpu.VMEM((per_worker,), jnp.int32),
            pltpu.VMEM((rows_pad,), jnp.int32),
            pltpu.VMEM((_LANES,), jnp.int32),
            pltpu.VMEM_SHARED((rows_pad,), jnp.int32),
            pltpu.SemaphoreType.DMA,
        ],
    )
    def body(logits_hbm, bids_hbm, out_hbm,
             rows_v, bids_v, out_v, tbl_v, res_v, tbl_sh, sem):
        cid = lax.axis_index("c")
        sid = lax.axis_index("s")
        wid = cid * _NUM_SUBCORES + sid
        base = wid * per_worker

        bids_cp = pltpu.async_copy(bids_hbm.at[pl.ds(base, per_worker)],
                                   bids_v, sem)

        row0 = sid * rows_per_sub
        pltpu.sync_copy(
            logits_hbm.at[pl.ds(row0 * vocab, rows_per_sub * vocab)], rows_v)
        lane = lax.iota(jnp.int32, _LANES)
        for g in range(row_groups):
            row_off = (lane + g * _LANES) * vocab

            def col_step(col, carry):
                best_val, best_idx = carry
                col_vec = jnp.full((_LANES,), col, jnp.int32)
                v = plsc.load_gather(rows_v, [row_off + col_vec])
                gt = v > best_val
                best_val = jnp.where(gt, v, best_val)
                best_idx = jnp.where(gt, col_vec, best_idx)
                return best_val, best_idx

            init = (jnp.full((_LANES,), -jnp.inf, jnp.float32),
                    jnp.zeros((_LANES,), jnp.int32))
            _, best_idx = lax.fori_loop(0, vocab, col_step, init, unroll=4)
            res_v[...] = best_idx
            pltpu.sync_copy(res_v, tbl_sh.at[pl.ds(row0 + g * _LANES, _LANES)])

        plsc.subcore_barrier()
        pltpu.sync_copy(tbl_sh, tbl_v)

        bids_cp.wait()
        max_row = jnp.full((_LANES,), max_id, jnp.int32)
        zero = jnp.zeros((_LANES,), jnp.int32)

        def idx_step(i, _):
            idx = bids_v[pl.ds(i * _LANES, _LANES)]
            idx = jnp.minimum(jnp.maximum(idx, zero), max_row)
            out_v[pl.ds(i * _LANES, _LANES)] = plsc.load_gather(tbl_v, [idx])
            return 0

        lax.fori_loop(0, n_idx_vecs, idx_step, 0, unroll=4)
        pltpu.sync_copy(out_v, out_hbm.at[pl.ds(base, per_worker)])

    return body(logits_pad.reshape(-1), bids_flat)


def kernel(byte_ids, logits):
    b, l = byte_ids.shape
    num_ids, vocab = logits.shape
    rows_pad = _NUM_WORKERS * _LANES
    logits_pad = jnp.pad(logits, ((0, rows_pad - num_ids), (0, 0)))
    bids_flat = byte_ids.reshape(-1).astype(jnp.int32)
    out = _lookup_call(bids_flat, logits_pad, rows_pad, vocab, num_ids - 1)
    return out.reshape(b, l)


# speedup vs baseline: 30.7384x; 1.1078x over previous
import functools

import jax
import jax.numpy as jnp
from jax import lax
from jax.experimental import pallas as pl
from jax.experimental.pallas import tpu as pltpu
from jax.experimental.pallas import tpu_sc as plsc

_LANES = 16
_NUM_CORES = 2
_NUM_SUBCORES = 16
_NUM_WORKERS = _NUM_CORES * _NUM_SUBCORES
_STREAMS = 4


def _lookup_call(bids_flat, logits_flat, num_ids, vocab):
    n = bids_flat.shape[0]
    per_worker = n // _NUM_WORKERS
    n_idx_vecs = per_worker // _LANES
    main_rows = _NUM_SUBCORES * _LANES
    extra_rows = range(main_rows, num_ids)
    tbl_len = main_rows + _LANES * len(extra_rows)
    seg = vocab // _STREAMS
    assert seg * _STREAMS == vocab and per_worker % _LANES == 0
    n_chunks = (vocab + _LANES - 1) // _LANES
    chunk_pad = n_chunks * _LANES

    mesh = plsc.VectorSubcoreMesh(core_axis_name="c", subcore_axis_name="s",
                                  num_cores=_NUM_CORES,
                                  num_subcores=_NUM_SUBCORES)

    @functools.partial(
        pl.kernel,
        mesh=mesh,
        out_type=jax.ShapeDtypeStruct((n,), jnp.int32),
        compiler_params=pltpu.CompilerParams(needs_layout_passes=False),
        scratch_types=[
            pltpu.VMEM((_LANES * vocab,), jnp.float32),
            pltpu.VMEM((chunk_pad,), jnp.float32),
            pltpu.VMEM((per_worker,), jnp.int32),
            pltpu.VMEM((per_worker,), jnp.int32),
            pltpu.VMEM((tbl_len,), jnp.int32),
            pltpu.VMEM((_LANES,), jnp.int32),
            pltpu.VMEM_SHARED((tbl_len,), jnp.int32),
            pltpu.SemaphoreType.DMA,
        ],
    )
    def body(logits_hbm, bids_hbm, out_hbm,
             rows_v, lrow_v, bids_v, out_v, tbl_v, res_v, tbl_sh, sem):
        cid = lax.axis_index("c")
        sid = lax.axis_index("s")
        wid = cid * _NUM_SUBCORES + sid
        base = wid * per_worker

        bids_cp = pltpu.async_copy(bids_hbm.at[pl.ds(base, per_worker)],
                                   bids_v, sem)

        pltpu.sync_copy(
            logits_hbm.at[pl.ds(sid * _LANES * vocab, _LANES * vocab)],
            rows_v)
        lane = lax.iota(jnp.int32, _LANES)
        neg_inf = jnp.full((_LANES,), -jnp.inf, jnp.float32)
        zero_i = jnp.zeros((_LANES,), jnp.int32)
        stream_base = [lane * vocab + k * seg for k in range(_STREAMS)]

        def col_step(c, accs):
            col_vec = jnp.full((_LANES,), c, jnp.int32)
            out = []
            for k in range(_STREAMS):
                best_val, best_idx = accs[k]
                v = plsc.load_gather(rows_v, [stream_base[k] + col_vec])
                gt = v > best_val
                out.append((jnp.where(gt, v, best_val),
                            jnp.where(gt, col_vec, best_idx)))
            return tuple(out)

        init = tuple((neg_inf, zero_i) for _ in range(_STREAMS))
        accs = lax.fori_loop(0, seg, col_step, init, unroll=2)
        best_val, best_idx = accs[0]
        for k in range(1, _STREAMS):
            v_k, i_k = accs[k]
            gt = v_k > best_val
            best_val = jnp.where(gt, v_k, best_val)
            best_idx = jnp.where(gt, i_k + (k * seg), best_idx)
        res_v[...] = best_idx
        pltpu.sync_copy(res_v, tbl_sh.at[pl.ds(sid * _LANES, _LANES)])

        @pl.when(sid == 0)
        def _():
            for j, row in enumerate(extra_rows):
                res_slot = main_rows + j * _LANES
                lrow_v[pl.ds(chunk_pad - _LANES, _LANES)] = neg_inf
                pltpu.sync_copy(logits_hbm.at[pl.ds(row * vocab, vocab)],
                                lrow_v.at[pl.ds(0, vocab)])

                def chunk_step(c, carry):
                    bv, bi = carry
                    v = lrow_v[pl.ds(c * _LANES, _LANES)]
                    gt = v > bv
                    return (jnp.where(gt, v, bv),
                            jnp.where(gt, jnp.full((_LANES,), c, jnp.int32),
                                      bi))

                bv, bi = lax.fori_loop(0, n_chunks, chunk_step,
                                       (neg_inf, zero_i), unroll=2)
                m = jnp.max(bv)
                col = bi * _LANES + lane
                cand = jnp.where(bv == m, col, jnp.full((_LANES,), vocab,
                                                        jnp.int32))
                res_v[...] = jnp.full((_LANES,), jnp.min(cand), jnp.int32)
                pltpu.sync_copy(res_v, tbl_sh.at[pl.ds(res_slot, _LANES)])

        plsc.subcore_barrier()
        pltpu.sync_copy(tbl_sh, tbl_v)

        bids_cp.wait()
        max_row = jnp.full((_LANES,), num_ids - 1, jnp.int32)

        def idx_step(i, _):
            idx = bids_v[pl.ds(i * _LANES, _LANES)]
            idx = jnp.minimum(jnp.maximum(idx, zero_i), max_row)
            out_v[pl.ds(i * _LANES, _LANES)] = plsc.load_gather(tbl_v, [idx])
            return 0

        lax.fori_loop(0, n_idx_vecs, idx_step, 0, unroll=4)
        pltpu.sync_copy(out_v, out_hbm.at[pl.ds(base, per_worker)])

    return body(logits_flat, bids_flat)


def kernel(byte_ids, logits):
    b, l = byte_ids.shape
    num_ids, vocab = logits.shape
    bids_flat = byte_ids.reshape(-1).astype(jnp.int32)
    out = _lookup_call(bids_flat, logits.reshape(-1), num_ids, vocab)
    return out.reshape(b, l)


# speedup vs baseline: 30.7919x; 1.0017x over previous
import functools

import jax
import jax.numpy as jnp
from jax import lax
from jax.experimental import pallas as pl
from jax.experimental.pallas import tpu as pltpu
from jax.experimental.pallas import tpu_sc as plsc

_LANES = 16
_NUM_CORES = 2
_NUM_SUBCORES = 16
_NUM_WORKERS = _NUM_CORES * _NUM_SUBCORES
_STREAMS = 4


def _lookup_call(bids_flat, logits_flat, num_ids, vocab):
    n = bids_flat.shape[0]
    per_worker = n // _NUM_WORKERS
    n_idx_vecs = per_worker // _LANES
    main_rows = _NUM_SUBCORES * _LANES
    extra_rows = range(main_rows, num_ids)
    tbl_len = main_rows + _LANES * len(extra_rows)
    seg = vocab // _STREAMS
    assert seg * _STREAMS == vocab and per_worker % _LANES == 0
    n_chunks = (vocab + _LANES - 1) // _LANES
    chunk_pad = n_chunks * _LANES

    mesh = plsc.VectorSubcoreMesh(core_axis_name="c", subcore_axis_name="s",
                                  num_cores=_NUM_CORES,
                                  num_subcores=_NUM_SUBCORES)

    @functools.partial(
        pl.kernel,
        mesh=mesh,
        out_type=jax.ShapeDtypeStruct((n,), jnp.int32),
        compiler_params=pltpu.CompilerParams(needs_layout_passes=False,
                                             skip_device_barrier=True),
        scratch_types=[
            pltpu.VMEM((_LANES * vocab,), jnp.float32),
            pltpu.VMEM((chunk_pad,), jnp.float32),
            pltpu.VMEM((per_worker,), jnp.int32),
            pltpu.VMEM((per_worker,), jnp.int32),
            pltpu.VMEM((tbl_len,), jnp.int32),
            pltpu.VMEM((_LANES,), jnp.int32),
            pltpu.VMEM_SHARED((tbl_len,), jnp.int32),
            pltpu.SemaphoreType.DMA,
        ],
    )
    def body(logits_hbm, bids_hbm, out_hbm,
             rows_v, lrow_v, bids_v, out_v, tbl_v, res_v, tbl_sh, sem):
        cid = lax.axis_index("c")
        sid = lax.axis_index("s")
        wid = cid * _NUM_SUBCORES + sid
        base = wid * per_worker

        bids_cp = pltpu.async_copy(bids_hbm.at[pl.ds(base, per_worker)],
                                   bids_v, sem)

        pltpu.sync_copy(
            logits_hbm.at[pl.ds(sid * _LANES * vocab, _LANES * vocab)],
            rows_v)
        lane = lax.iota(jnp.int32, _LANES)
        neg_inf = jnp.full((_LANES,), -jnp.inf, jnp.float32)
        zero_i = jnp.zeros((_LANES,), jnp.int32)
        stream_base = [lane * vocab + k * seg for k in range(_STREAMS)]

        def col_step(c, accs):
            col_vec = jnp.full((_LANES,), c, jnp.int32)
            out = []
            for k in range(_STREAMS):
                best_val, best_idx = accs[k]
                v = plsc.load_gather(rows_v, [stream_base[k] + col_vec])
                gt = v > best_val
                out.append((jnp.where(gt, v, best_val),
                            jnp.where(gt, col_vec, best_idx)))
            return tuple(out)

        init = tuple((neg_inf, zero_i) for _ in range(_STREAMS))
        accs = lax.fori_loop(0, seg, col_step, init, unroll=2)
        best_val, best_idx = accs[0]
        for k in range(1, _STREAMS):
            v_k, i_k = accs[k]
            gt = v_k > best_val
            best_val = jnp.where(gt, v_k, best_val)
            best_idx = jnp.where(gt, i_k + (k * seg), best_idx)
        res_v[...] = best_idx
        pltpu.sync_copy(res_v, tbl_sh.at[pl.ds(sid * _LANES, _LANES)])

        @pl.when(sid == 0)
        def _():
            for j, row in enumerate(extra_rows):
                res_slot = main_rows + j * _LANES
                lrow_v[pl.ds(chunk_pad - _LANES, _LANES)] = neg_inf
                pltpu.sync_copy(logits_hbm.at[pl.ds(row * vocab, vocab)],
                                lrow_v.at[pl.ds(0, vocab)])

                def chunk_step(c, carry):
                    bv, bi = carry
                    v = lrow_v[pl.ds(c * _LANES, _LANES)]
                    gt = v > bv
                    return (jnp.where(gt, v, bv),
                            jnp.where(gt, jnp.full((_LANES,), c, jnp.int32),
                                      bi))

                bv, bi = lax.fori_loop(0, n_chunks, chunk_step,
                                       (neg_inf, zero_i), unroll=2)
                m = jnp.max(bv)
                col = bi * _LANES + lane
                cand = jnp.where(bv == m, col, jnp.full((_LANES,), vocab,
                                                        jnp.int32))
                res_v[...] = jnp.full((_LANES,), jnp.min(cand), jnp.int32)
                pltpu.sync_copy(res_v, tbl_sh.at[pl.ds(res_slot, _LANES)])

        plsc.subcore_barrier()
        pltpu.sync_copy(tbl_sh, tbl_v)

        bids_cp.wait()
        max_row = jnp.full((_LANES,), num_ids - 1, jnp.int32)

        def idx_step(i, _):
            idx = bids_v[pl.ds(i * _LANES, _LANES)]
            idx = jnp.minimum(jnp.maximum(idx, zero_i), max_row)
            out_v[pl.ds(i * _LANES, _LANES)] = plsc.load_gather(tbl_v, [idx])
            return 0

        lax.fori_loop(0, n_idx_vecs, idx_step, 0, unroll=4)
        pltpu.sync_copy(out_v, out_hbm.at[pl.ds(base, per_worker)])

    return body(logits_flat, bids_flat)


def kernel(byte_ids, logits):
    b, l = byte_ids.shape
    num_ids, vocab = logits.shape
    bids_flat = byte_ids.reshape(-1).astype(jnp.int32)
    out = _lookup_call(bids_flat, logits.reshape(-1), num_ids, vocab)
    return out.reshape(b, l)
